# Initial kernel scaffold; baseline (speedup 1.0000x reference)
import jax
import jax.numpy as jnp
from jax.experimental import pallas as pl


def kernel(x, edge_index, W, b, gamma, beta):
    raise NotImplementedError("write your pallas kernel here")


# speedup vs baseline: 9.7310x; 9.7310x over previous
import functools

import jax
import jax.numpy as jnp
from jax import lax
from jax.experimental import pallas as pl
from jax.experimental.pallas import tpu as pltpu
from jax.experimental.pallas import tpu_sc as plsc

N = 10000
D = 128
E = 320000
NC = 2
NS = 16
LANES = 16
NW = NC * NS
CHUNK = 128
CPW = 80
EPW = CPW * CHUNK
E_PAD = NW * EPW
N_ACC = 10240
RPS = N_ACC // NS
BLK = 1000


def _mesh():
    return plsc.VectorSubcoreMesh(
        core_axis_name="c", subcore_axis_name="s",
        num_cores=NC, num_subcores=NS)


def _hist_body(dst_hbm, out_hbm, idxbuf, deg):
    c = lax.axis_index("c")
    s = lax.axis_index("s")
    w = c * NS + s

    def zero(i, _):
        deg[pl.ds(i * LANES, LANES)] = jnp.zeros((LANES,), jnp.float32)
        return 0
    lax.fori_loop(0, N_ACC // LANES, zero, 0)

    pltpu.sync_copy(dst_hbm.at[pl.ds(w * CPW, CPW)], idxbuf)
    ones = jnp.ones((LANES,), jnp.float32)

    def chunk(j, _):
        def sub(k, _2):
            idx = idxbuf[j, pl.ds(k * LANES, LANES)]
            plsc.addupdate_scatter(deg, [idx], ones)
            return 0
        lax.fori_loop(0, CHUNK // LANES, sub, 0)
        return 0
    lax.fori_loop(0, CPW, chunk, 0)

    pltpu.sync_copy(deg, out_hbm.at[w])


_hist = functools.partial(
    pl.kernel,
    out_type=jax.ShapeDtypeStruct((NW, N_ACC), jnp.float32),
    mesh=_mesh(),
    scratch_types=[
        pltpu.VMEM((CPW, CHUNK), jnp.int32),
        pltpu.VMEM((N_ACC,), jnp.float32),
    ],
    compiler_params=pltpu.CompilerParams(needs_layout_passes=False),
)(_hist_body)


def _scat_body(y_hbm, src_hbm, dst_hbm, out_hbm, sbuf, dbuf, rows, acc, sem):
    c = lax.axis_index("c")
    s = lax.axis_index("s")
    w = c * NS + s

    def zero(i, _):
        for jj in range(D // LANES):
            rows[i, pl.ds(jj * LANES, LANES)] = jnp.zeros((LANES,), jnp.float32)
        return 0
    lax.fori_loop(0, CHUNK, zero, 0)
    for k in range(RPS // CHUNK):
        pltpu.sync_copy(rows, acc.at[pl.ds(s * RPS + k * CHUNK, CHUNK)])

    pltpu.sync_copy(src_hbm.at[pl.ds(w * CPW, CPW)], sbuf)
    pltpu.sync_copy(dst_hbm.at[pl.ds(w * CPW, CPW)], dbuf)
    plsc.subcore_barrier()

    def chunk(j, _):
        pltpu.async_copy(y_hbm.at[sbuf.at[j]], rows, sem).wait()
        pltpu.sync_copy(rows, acc.at[dbuf.at[j]], add=True)
        return 0
    lax.fori_loop(0, CPW, chunk, 0)

    plsc.subcore_barrier()
    pltpu.sync_copy(acc.at[pl.ds(s * RPS, RPS)],
                    out_hbm.at[c, pl.ds(s * RPS, RPS)])


_scat = functools.partial(
    pl.kernel,
    out_type=jax.ShapeDtypeStruct((NC, N_ACC, D), jnp.float32),
    mesh=_mesh(),
    scratch_types=[
        pltpu.VMEM((CPW, CHUNK), jnp.int32),
        pltpu.VMEM((CPW, CHUNK), jnp.int32),
        pltpu.VMEM((CHUNK, D), jnp.float32),
        pltpu.VMEM_SHARED((N_ACC, D), jnp.float32),
        pltpu.SemaphoreType.DMA,
    ],
)(_scat_body)


def _mid_body(x_ref, w_ref, degp_ref, y_ref):
    xw = jnp.dot(x_ref[...], w_ref[...], preferred_element_type=jnp.float32)
    deg = jnp.sum(degp_ref[...], axis=0) + 1.0
    y_ref[...] = xw * lax.rsqrt(deg)


_mid = pl.pallas_call(
    _mid_body,
    grid=(N // BLK,),
    in_specs=[
        pl.BlockSpec((BLK, D), lambda i: (i, 0)),
        pl.BlockSpec((D, D), lambda i: (0, 0)),
        pl.BlockSpec((NW, BLK, 1), lambda i: (0, i, 0)),
    ],
    out_specs=pl.BlockSpec((BLK, D), lambda i: (i, 0)),
    out_shape=jax.ShapeDtypeStruct((N, D), jnp.float32),
)


def _epi_body(aggp_ref, y_ref, x_ref, degp_ref, b_ref, g_ref, bt_ref, o_ref):
    deg = jnp.sum(degp_ref[...], axis=0) + 1.0
    dinv = lax.rsqrt(deg)
    t = dinv * (aggp_ref[0] + aggp_ref[1] + y_ref[...]) + b_ref[...] + x_ref[...]
    mu = jnp.mean(t, axis=-1, keepdims=True)
    var = jnp.mean((t - mu) ** 2, axis=-1, keepdims=True)
    t = (t - mu) * lax.rsqrt(var + 1e-5) * g_ref[...] + bt_ref[...]
    o_ref[...] = jnp.maximum(t, 0.0)


_epi = pl.pallas_call(
    _epi_body,
    grid=(N // BLK,),
    in_specs=[
        pl.BlockSpec((NC, BLK, D), lambda i: (0, i, 0)),
        pl.BlockSpec((BLK, D), lambda i: (i, 0)),
        pl.BlockSpec((BLK, D), lambda i: (i, 0)),
        pl.BlockSpec((NW, BLK, 1), lambda i: (0, i, 0)),
        pl.BlockSpec((1, D), lambda i: (0, 0)),
        pl.BlockSpec((1, D), lambda i: (0, 0)),
        pl.BlockSpec((1, D), lambda i: (0, 0)),
    ],
    out_specs=pl.BlockSpec((BLK, D), lambda i: (i, 0)),
    out_shape=jax.ShapeDtypeStruct((N, D), jnp.float32),
)


def kernel(x, edge_index, W, b, gamma, beta):
    src = edge_index[0]
    dst = edge_index[1]
    pad = E_PAD - E
    src_p = jnp.concatenate(
        [src, jnp.zeros((pad,), jnp.int32)]).reshape(NW * CPW, CHUNK)
    dst_p = jnp.concatenate(
        [dst, jnp.full((pad,), N, jnp.int32)]).reshape(NW * CPW, CHUNK)
    degp = _hist(dst_p).reshape(NW, N_ACC, 1)
    y = _mid(x, W, degp)
    aggp = _scat(y, src_p, dst_p)
    return _epi(aggp, y, x, degp,
                b.reshape(1, D), gamma.reshape(1, D), beta.reshape(1, D))


# speedup vs baseline: 10.6777x; 1.0973x over previous
import functools

import jax
import jax.numpy as jnp
from jax import lax
from jax.experimental import pallas as pl
from jax.experimental.pallas import tpu as pltpu
from jax.experimental.pallas import tpu_sc as plsc

N = 10000
D = 128
E = 320000
NC = 2
NS = 16
LANES = 16
NW = NC * NS
CHUNK = 128
CPW = 80
STAGE = 40
EPW = CPW * CHUNK
E_PAD = NW * EPW
N_ACC = 10240
RPS = N_ACC // NS
BLK = 1000


def _mesh():
    return plsc.VectorSubcoreMesh(
        core_axis_name="c", subcore_axis_name="s",
        num_cores=NC, num_subcores=NS)


def _hist_body(dst_hbm, out_hbm, idxbuf, deg):
    c = lax.axis_index("c")
    s = lax.axis_index("s")
    w = c * NS + s

    def zero(i, _):
        deg[pl.ds(i * LANES, LANES)] = jnp.zeros((LANES,), jnp.float32)
        return 0
    lax.fori_loop(0, N_ACC // LANES, zero, 0)

    pltpu.sync_copy(dst_hbm.at[pl.ds(w * CPW, CPW)], idxbuf)
    ones = jnp.ones((LANES,), jnp.float32)

    def chunk(j, _):
        def sub(k, _2):
            idx = idxbuf[j, pl.ds(k * LANES, LANES)]
            plsc.addupdate_scatter(deg, [idx], ones)
            return 0
        lax.fori_loop(0, CHUNK // LANES, sub, 0)
        return 0
    lax.fori_loop(0, CPW, chunk, 0)

    pltpu.sync_copy(deg, out_hbm.at[w])


_hist = functools.partial(
    pl.kernel,
    out_type=jax.ShapeDtypeStruct((NW, N_ACC), jnp.float32),
    mesh=_mesh(),
    scratch_types=[
        pltpu.VMEM((CPW, CHUNK), jnp.int32),
        pltpu.VMEM((N_ACC,), jnp.float32),
    ],
    compiler_params=pltpu.CompilerParams(needs_layout_passes=False),
)(_hist_body)


def _scat_body(y_hbm, src_hbm, dst_hbm, out_hbm, sbuf, dbuf, rows_a, rows_b,
               acc, sem_a, sem_b):
    c = lax.axis_index("c")
    s = lax.axis_index("s")
    w = c * NS + s

    def zero(i, _):
        for jj in range(D // LANES):
            rows_a[i, pl.ds(jj * LANES, LANES)] = jnp.zeros((LANES,),
                                                            jnp.float32)
        return 0
    lax.fori_loop(0, CHUNK, zero, 0)
    for k in range(RPS // CHUNK):
        pltpu.sync_copy(rows_a, acc.at[pl.ds(s * RPS + k * CHUNK, CHUNK)])

    plsc.subcore_barrier()

    def gather(j, buf, sem):
        pltpu.async_copy(y_hbm.at[sbuf.at[j]], buf, sem)

    def drain(j, buf, sem):
        pltpu.make_async_copy(y_hbm.at[sbuf.at[j]], buf, sem).wait()

    def scat(j, buf):
        pltpu.sync_copy(buf, acc.at[dbuf.at[j]], add=True)

    for h in range(CPW // STAGE):
        base = w * CPW + h * STAGE
        pltpu.sync_copy(src_hbm.at[pl.ds(base, STAGE)], sbuf)
        pltpu.sync_copy(dst_hbm.at[pl.ds(base, STAGE)], dbuf)
        gather(0, rows_a, sem_a)

        def pair(i, _):
            ja = 2 * i
            jb = 2 * i + 1
            gather(jb, rows_b, sem_b)
            drain(ja, rows_a, sem_a)
            scat(ja, rows_a)
            gather(ja + 2, rows_a, sem_a)
            drain(jb, rows_b, sem_b)
            scat(jb, rows_b)
            return 0
        lax.fori_loop(0, (STAGE - 2) // 2, pair, 0)

        gather(STAGE - 1, rows_b, sem_b)
        drain(STAGE - 2, rows_a, sem_a)
        scat(STAGE - 2, rows_a)
        drain(STAGE - 1, rows_b, sem_b)
        scat(STAGE - 1, rows_b)

    plsc.subcore_barrier()
    pltpu.sync_copy(acc.at[pl.ds(s * RPS, RPS)],
                    out_hbm.at[c, pl.ds(s * RPS, RPS)])


_scat = functools.partial(
    pl.kernel,
    out_type=jax.ShapeDtypeStruct((NC, N_ACC, D), jnp.float32),
    mesh=_mesh(),
    scratch_types=[
        pltpu.VMEM((STAGE, CHUNK), jnp.int32),
        pltpu.VMEM((STAGE, CHUNK), jnp.int32),
        pltpu.VMEM((CHUNK, D), jnp.float32),
        pltpu.VMEM((CHUNK, D), jnp.float32),
        pltpu.VMEM_SHARED((N_ACC, D), jnp.float32),
        pltpu.SemaphoreType.DMA,
        pltpu.SemaphoreType.DMA,
    ],
)(_scat_body)


def _mid_body(x_ref, w_ref, degp_ref, y_ref):
    xw = jnp.dot(x_ref[...], w_ref[...], preferred_element_type=jnp.float32)
    deg = jnp.sum(degp_ref[...], axis=0) + 1.0
    y_ref[...] = xw * lax.rsqrt(deg)


_mid = pl.pallas_call(
    _mid_body,
    grid=(N // BLK,),
    in_specs=[
        pl.BlockSpec((BLK, D), lambda i: (i, 0)),
        pl.BlockSpec((D, D), lambda i: (0, 0)),
        pl.BlockSpec((NW, BLK, 1), lambda i: (0, i, 0)),
    ],
    out_specs=pl.BlockSpec((BLK, D), lambda i: (i, 0)),
    out_shape=jax.ShapeDtypeStruct((N, D), jnp.float32),
)


def _epi_body(aggp_ref, y_ref, x_ref, degp_ref, b_ref, g_ref, bt_ref, o_ref):
    deg = jnp.sum(degp_ref[...], axis=0) + 1.0
    dinv = lax.rsqrt(deg)
    t = dinv * (aggp_ref[0] + aggp_ref[1] + y_ref[...]) + b_ref[...] + x_ref[...]
    mu = jnp.mean(t, axis=-1, keepdims=True)
    var = jnp.mean((t - mu) ** 2, axis=-1, keepdims=True)
    t = (t - mu) * lax.rsqrt(var + 1e-5) * g_ref[...] + bt_ref[...]
    o_ref[...] = jnp.maximum(t, 0.0)


_epi = pl.pallas_call(
    _epi_body,
    grid=(N // BLK,),
    in_specs=[
        pl.BlockSpec((NC, BLK, D), lambda i: (0, i, 0)),
        pl.BlockSpec((BLK, D), lambda i: (i, 0)),
        pl.BlockSpec((BLK, D), lambda i: (i, 0)),
        pl.BlockSpec((NW, BLK, 1), lambda i: (0, i, 0)),
        pl.BlockSpec((1, D), lambda i: (0, 0)),
        pl.BlockSpec((1, D), lambda i: (0, 0)),
        pl.BlockSpec((1, D), lambda i: (0, 0)),
    ],
    out_specs=pl.BlockSpec((BLK, D), lambda i: (i, 0)),
    out_shape=jax.ShapeDtypeStruct((N, D), jnp.float32),
)


def kernel(x, edge_index, W, b, gamma, beta):
    src = edge_index[0]
    dst = edge_index[1]
    pad = E_PAD - E
    src_p = jnp.concatenate(
        [src, jnp.zeros((pad,), jnp.int32)]).reshape(NW * CPW, CHUNK)
    dst_p = jnp.concatenate(
        [dst, jnp.full((pad,), N, jnp.int32)]).reshape(NW * CPW, CHUNK)
    degp = _hist(dst_p).reshape(NW, N_ACC, 1)
    y = _mid(x, W, degp)
    aggp = _scat(y, src_p, dst_p)
    return _epi(aggp, y, x, degp,
                b.reshape(1, D), gamma.reshape(1, D), beta.reshape(1, D))


# speedup vs baseline: 11.0135x; 1.0314x over previous
import functools

import jax
import jax.numpy as jnp
from jax import lax
from jax.experimental import pallas as pl
from jax.experimental.pallas import tpu as pltpu
from jax.experimental.pallas import tpu_sc as plsc

N = 10000
D = 128
E = 320000
NC = 2
NS = 16
LANES = 16
NW = NC * NS
CHUNK = 128
CPW = 80
CPW0 = 128
CPW1 = 32
STAGE = 32
EPW = CPW * CHUNK
E_PAD = NW * EPW
N_ACC = 10240
RPS = N_ACC // NS
BLK = 1000


def _mesh():
    return plsc.VectorSubcoreMesh(
        core_axis_name="c", subcore_axis_name="s",
        num_cores=NC, num_subcores=NS)


def _hist_body(dst_hbm, out_hbm, idxbuf, deg):
    c = lax.axis_index("c")
    s = lax.axis_index("s")
    w = c * NS + s

    def zero(i, _):
        deg[pl.ds(i * LANES, LANES)] = jnp.zeros((LANES,), jnp.float32)
        return 0
    lax.fori_loop(0, N_ACC // LANES, zero, 0)

    pltpu.sync_copy(dst_hbm.at[pl.ds(w * CPW, CPW)], idxbuf)
    ones = jnp.ones((LANES,), jnp.float32)

    def chunk(j, _):
        def sub(k, _2):
            idx = idxbuf[j, pl.ds(k * LANES, LANES)]
            plsc.addupdate_scatter(deg, [idx], ones)
            return 0
        lax.fori_loop(0, CHUNK // LANES, sub, 0)
        return 0
    lax.fori_loop(0, CPW, chunk, 0)

    pltpu.sync_copy(deg, out_hbm.at[w])


_hist = functools.partial(
    pl.kernel,
    out_type=jax.ShapeDtypeStruct((NW, N_ACC), jnp.float32),
    mesh=_mesh(),
    scratch_types=[
        pltpu.VMEM((CPW, CHUNK), jnp.int32),
        pltpu.VMEM((N_ACC,), jnp.float32),
    ],
    compiler_params=pltpu.CompilerParams(needs_layout_passes=False),
)(_hist_body)


def _scat_body(y_hbm, src_hbm, dst_hbm, out_hbm, sbuf, dbuf, rows_a, rows_b,
               acc, sem_a, sem_b):
    c = lax.axis_index("c")
    s = lax.axis_index("s")
    w = c * NS + s

    def zero(i, _):
        for jj in range(D // LANES):
            rows_a[i, pl.ds(jj * LANES, LANES)] = jnp.zeros((LANES,),
                                                            jnp.float32)
        return 0
    lax.fori_loop(0, CHUNK, zero, 0)
    for k in range(RPS // CHUNK):
        pltpu.sync_copy(rows_a, acc.at[pl.ds(s * RPS + k * CHUNK, CHUNK)])

    plsc.subcore_barrier()

    def gather(j, buf, sem):
        pltpu.async_copy(y_hbm.at[sbuf.at[j]], buf, sem)

    def drain(j, buf, sem):
        pltpu.make_async_copy(y_hbm.at[sbuf.at[j]], buf, sem).wait()

    def scat(j, buf):
        pltpu.sync_copy(buf, acc.at[dbuf.at[j]], add=True)

    def pipeline(row0, n_chunks):
        for h in range(n_chunks // STAGE):
            base = row0 + h * STAGE
            pltpu.sync_copy(src_hbm.at[pl.ds(base, STAGE)], sbuf)
            pltpu.sync_copy(dst_hbm.at[pl.ds(base, STAGE)], dbuf)
            gather(0, rows_a, sem_a)

            def pair(i, _):
                ja = 2 * i
                jb = 2 * i + 1
                gather(jb, rows_b, sem_b)
                drain(ja, rows_a, sem_a)
                scat(ja, rows_a)
                gather(ja + 2, rows_a, sem_a)
                drain(jb, rows_b, sem_b)
                scat(jb, rows_b)
                return 0
            lax.fori_loop(0, (STAGE - 2) // 2, pair, 0)

            gather(STAGE - 1, rows_b, sem_b)
            drain(STAGE - 2, rows_a, sem_a)
            scat(STAGE - 2, rows_a)
            drain(STAGE - 1, rows_b, sem_b)
            scat(STAGE - 1, rows_b)

    @pl.when(c == 0)
    def _():
        pipeline(s * CPW0, CPW0)

    @pl.when(c == 1)
    def _():
        pipeline(NS * CPW0 + s * CPW1, CPW1)

    plsc.subcore_barrier()
    pltpu.sync_copy(acc.at[pl.ds(s * RPS, RPS)],
                    out_hbm.at[c, pl.ds(s * RPS, RPS)])


_scat = functools.partial(
    pl.kernel,
    out_type=jax.ShapeDtypeStruct((NC, N_ACC, D), jnp.float32),
    mesh=_mesh(),
    scratch_types=[
        pltpu.VMEM((STAGE, CHUNK), jnp.int32),
        pltpu.VMEM((STAGE, CHUNK), jnp.int32),
        pltpu.VMEM((CHUNK, D), jnp.float32),
        pltpu.VMEM((CHUNK, D), jnp.float32),
        pltpu.VMEM_SHARED((N_ACC, D), jnp.float32),
        pltpu.SemaphoreType.DMA,
        pltpu.SemaphoreType.DMA,
    ],
)(_scat_body)


def _mid_body(x_ref, w_ref, degp_ref, y_ref):
    xw = jnp.dot(x_ref[...], w_ref[...], preferred_element_type=jnp.float32)
    deg = jnp.sum(degp_ref[...], axis=0) + 1.0
    y_ref[...] = xw * lax.rsqrt(deg)


_mid = pl.pallas_call(
    _mid_body,
    grid=(N // BLK,),
    in_specs=[
        pl.BlockSpec((BLK, D), lambda i: (i, 0)),
        pl.BlockSpec((D, D), lambda i: (0, 0)),
        pl.BlockSpec((NW, BLK, 1), lambda i: (0, i, 0)),
    ],
    out_specs=pl.BlockSpec((BLK, D), lambda i: (i, 0)),
    out_shape=jax.ShapeDtypeStruct((N, D), jnp.float32),
)


def _epi_body(aggp_ref, y_ref, x_ref, degp_ref, b_ref, g_ref, bt_ref, o_ref):
    deg = jnp.sum(degp_ref[...], axis=0) + 1.0
    dinv = lax.rsqrt(deg)
    t = dinv * (aggp_ref[0] + aggp_ref[1] + y_ref[...]) + b_ref[...] + x_ref[...]
    mu = jnp.mean(t, axis=-1, keepdims=True)
    var = jnp.mean((t - mu) ** 2, axis=-1, keepdims=True)
    t = (t - mu) * lax.rsqrt(var + 1e-5) * g_ref[...] + bt_ref[...]
    o_ref[...] = jnp.maximum(t, 0.0)


_epi = pl.pallas_call(
    _epi_body,
    grid=(N // BLK,),
    in_specs=[
        pl.BlockSpec((NC, BLK, D), lambda i: (0, i, 0)),
        pl.BlockSpec((BLK, D), lambda i: (i, 0)),
        pl.BlockSpec((BLK, D), lambda i: (i, 0)),
        pl.BlockSpec((NW, BLK, 1), lambda i: (0, i, 0)),
        pl.BlockSpec((1, D), lambda i: (0, 0)),
        pl.BlockSpec((1, D), lambda i: (0, 0)),
        pl.BlockSpec((1, D), lambda i: (0, 0)),
    ],
    out_specs=pl.BlockSpec((BLK, D), lambda i: (i, 0)),
    out_shape=jax.ShapeDtypeStruct((N, D), jnp.float32),
)


def kernel(x, edge_index, W, b, gamma, beta):
    src = edge_index[0]
    dst = edge_index[1]
    pad = E_PAD - E
    src_p = jnp.concatenate(
        [src, jnp.zeros((pad,), jnp.int32)]).reshape(NW * CPW, CHUNK)
    dst_p = jnp.concatenate(
        [dst, jnp.full((pad,), N, jnp.int32)]).reshape(NW * CPW, CHUNK)
    degp = _hist(dst_p).reshape(NW, N_ACC, 1)
    y = _mid(x, W, degp)
    aggp = _scat(y, src_p, dst_p)
    return _epi(aggp, y, x, degp,
                b.reshape(1, D), gamma.reshape(1, D), beta.reshape(1, D))


# speedup vs baseline: 12.7908x; 1.1614x over previous
import functools

import jax
import jax.numpy as jnp
from jax import lax
from jax.experimental import pallas as pl
from jax.experimental.pallas import tpu as pltpu
from jax.experimental.pallas import tpu_sc as plsc

N = 10000
D = 128
E = 320000
NC = 2
NS = 16
LANES = 16
NW = NC * NS
CHUNK = 128
CPW = 80
CPW0 = 128
CPW1 = 32
STAGE = 32
EPW = CPW * CHUNK
E_PAD = NW * EPW
N_ACC = 10240
RPS = N_ACC // NS
BLK = 1000


def _mesh():
    return plsc.VectorSubcoreMesh(
        core_axis_name="c", subcore_axis_name="s",
        num_cores=NC, num_subcores=NS)


def _hist_body(dst_hbm, out_hbm, idxbuf, deg):
    c = lax.axis_index("c")
    s = lax.axis_index("s")
    w = c * NS + s

    def zero(i, _):
        deg[pl.ds(i * LANES, LANES)] = jnp.zeros((LANES,), jnp.float32)
        return 0
    lax.fori_loop(0, N_ACC // LANES, zero, 0)

    pltpu.sync_copy(dst_hbm.at[pl.ds(w * CPW, CPW)], idxbuf)
    ones = jnp.ones((LANES,), jnp.float32)

    def chunk(j, _):
        def sub(k, _2):
            idx = idxbuf[j, pl.ds(k * LANES, LANES)]
            plsc.addupdate_scatter(deg, [idx], ones)
            return 0
        lax.fori_loop(0, CHUNK // LANES, sub, 0)
        return 0
    lax.fori_loop(0, CPW, chunk, 0)

    pltpu.sync_copy(deg, out_hbm.at[w])


_hist = functools.partial(
    pl.kernel,
    out_type=jax.ShapeDtypeStruct((NW, N_ACC), jnp.float32),
    mesh=_mesh(),
    scratch_types=[
        pltpu.VMEM((CPW, CHUNK), jnp.int32),
        pltpu.VMEM((N_ACC,), jnp.float32),
    ],
    compiler_params=pltpu.CompilerParams(needs_layout_passes=False),
)(_hist_body)


def _scat_body(y_hbm, src_hbm, dst_hbm, out_hbm, sbuf, dbuf, rows_a, rows_b,
               acc, sem_a, sem_b):
    c = lax.axis_index("c")
    s = lax.axis_index("s")
    w = c * NS + s

    def zero(i, _):
        for jj in range(D // LANES):
            rows_a[i, pl.ds(jj * LANES, LANES)] = jnp.zeros((LANES,),
                                                            jnp.float32)
        return 0
    lax.fori_loop(0, CHUNK, zero, 0)
    for k in range(RPS // CHUNK):
        pltpu.sync_copy(rows_a, acc.at[pl.ds(s * RPS + k * CHUNK, CHUNK)])

    plsc.subcore_barrier()

    def gather(j, buf, sem):
        pltpu.async_copy(y_hbm.at[sbuf.at[j]], buf, sem)

    def drain(j, buf, sem):
        pltpu.make_async_copy(y_hbm.at[sbuf.at[j]], buf, sem).wait()

    def scat(j, buf):
        pltpu.sync_copy(buf, acc.at[dbuf.at[j]], add=True)

    def pipeline(row0, n_chunks):
        for h in range(n_chunks // STAGE):
            base = row0 + h * STAGE
            pltpu.sync_copy(src_hbm.at[pl.ds(base, STAGE)], sbuf)
            pltpu.sync_copy(dst_hbm.at[pl.ds(base, STAGE)], dbuf)
            gather(0, rows_a, sem_a)

            def pair(i, _):
                ja = 2 * i
                jb = 2 * i + 1
                gather(jb, rows_b, sem_b)
                drain(ja, rows_a, sem_a)
                scat(ja, rows_a)
                gather(ja + 2, rows_a, sem_a)
                drain(jb, rows_b, sem_b)
                scat(jb, rows_b)
                return 0
            lax.fori_loop(0, (STAGE - 2) // 2, pair, 0)

            gather(STAGE - 1, rows_b, sem_b)
            drain(STAGE - 2, rows_a, sem_a)
            scat(STAGE - 2, rows_a)
            drain(STAGE - 1, rows_b, sem_b)
            scat(STAGE - 1, rows_b)

    @pl.when(c == 0)
    def _():
        pipeline(s * CPW0, CPW0)

    @pl.when(c == 1)
    def _():
        pipeline(NS * CPW0 + s * CPW1, CPW1)

    plsc.subcore_barrier()
    pltpu.sync_copy(acc.at[pl.ds(s * RPS, RPS)],
                    out_hbm.at[c, pl.ds(s * RPS, RPS)])


_scat = functools.partial(
    pl.kernel,
    out_type=jax.ShapeDtypeStruct((NC, N_ACC, D), jnp.float32),
    mesh=_mesh(),
    scratch_types=[
        pltpu.VMEM((STAGE, CHUNK), jnp.int32),
        pltpu.VMEM((STAGE, CHUNK), jnp.int32),
        pltpu.VMEM((CHUNK, D), jnp.float32),
        pltpu.VMEM((CHUNK, D), jnp.float32),
        pltpu.VMEM_SHARED((N_ACC, D), jnp.float32),
        pltpu.SemaphoreType.DMA,
        pltpu.SemaphoreType.DMA,
    ],
)(_scat_body)


def _mid_body(x_ref, w_ref, degp_ref, y_ref):
    xw = jnp.dot(x_ref[...], w_ref[...], preferred_element_type=jnp.float32)
    deg = jnp.sum(degp_ref[...], axis=1, keepdims=True) + 1.0
    y_ref[...] = xw * lax.rsqrt(deg)


_mid = pl.pallas_call(
    _mid_body,
    grid=(N // BLK,),
    in_specs=[
        pl.BlockSpec((BLK, D), lambda i: (i, 0)),
        pl.BlockSpec((D, D), lambda i: (0, 0)),
        pl.BlockSpec((BLK, NW), lambda i: (i, 0)),
    ],
    out_specs=pl.BlockSpec((BLK, D), lambda i: (i, 0)),
    out_shape=jax.ShapeDtypeStruct((N, D), jnp.float32),
)


def _epi_body(aggp_ref, y_ref, x_ref, degp_ref, b_ref, g_ref, bt_ref, o_ref):
    deg = jnp.sum(degp_ref[...], axis=1, keepdims=True) + 1.0
    dinv = lax.rsqrt(deg)
    t = dinv * (aggp_ref[0] + aggp_ref[1] + y_ref[...]) + b_ref[...] + x_ref[...]
    mu = jnp.mean(t, axis=-1, keepdims=True)
    var = jnp.mean((t - mu) ** 2, axis=-1, keepdims=True)
    t = (t - mu) * lax.rsqrt(var + 1e-5) * g_ref[...] + bt_ref[...]
    o_ref[...] = jnp.maximum(t, 0.0)


_epi = pl.pallas_call(
    _epi_body,
    grid=(N // BLK,),
    in_specs=[
        pl.BlockSpec((NC, BLK, D), lambda i: (0, i, 0)),
        pl.BlockSpec((BLK, D), lambda i: (i, 0)),
        pl.BlockSpec((BLK, D), lambda i: (i, 0)),
        pl.BlockSpec((BLK, NW), lambda i: (i, 0)),
        pl.BlockSpec((1, D), lambda i: (0, 0)),
        pl.BlockSpec((1, D), lambda i: (0, 0)),
        pl.BlockSpec((1, D), lambda i: (0, 0)),
    ],
    out_specs=pl.BlockSpec((BLK, D), lambda i: (i, 0)),
    out_shape=jax.ShapeDtypeStruct((N, D), jnp.float32),
)


def kernel(x, edge_index, W, b, gamma, beta):
    src = edge_index[0]
    dst = edge_index[1]
    pad = E_PAD - E
    src_p = jnp.concatenate(
        [src, jnp.zeros((pad,), jnp.int32)]).reshape(NW * CPW, CHUNK)
    dst_p = jnp.concatenate(
        [dst, jnp.full((pad,), N, jnp.int32)]).reshape(NW * CPW, CHUNK)
    degp = _hist(dst_p).T
    y = _mid(x, W, degp)
    aggp = _scat(y, src_p, dst_p)
    return _epi(aggp, y, x, degp,
                b.reshape(1, D), gamma.reshape(1, D), beta.reshape(1, D))


# speedup vs baseline: 15.1036x; 1.1808x over previous
import functools

import jax
import jax.numpy as jnp
from jax import lax
from jax.experimental import pallas as pl
from jax.experimental.pallas import tpu as pltpu
from jax.experimental.pallas import tpu_sc as plsc

N = 10000
D = 128
E = 320000
NC = 2
NS = 16
LANES = 16
NW = NC * NS
CHUNK = 128
CPW = 80
CPW0 = 144
CPW1 = 16
STAGE = 32
EPW = CPW * CHUNK
E_PAD = NW * EPW
N_ACC = 10240
RPS = N_ACC // NS
BLK = 1000


def _mesh():
    return plsc.VectorSubcoreMesh(
        core_axis_name="c", subcore_axis_name="s",
        num_cores=NC, num_subcores=NS)


def _hist_body(dst_hbm, out_hbm, idxbuf, deg):
    c = lax.axis_index("c")
    s = lax.axis_index("s")
    w = c * NS + s

    def zero(i, _):
        deg[pl.ds(i * LANES, LANES)] = jnp.zeros((LANES,), jnp.float32)
        return 0
    lax.fori_loop(0, N_ACC // LANES, zero, 0)

    pltpu.sync_copy(dst_hbm.at[pl.ds(w * CPW, CPW)], idxbuf)
    ones = jnp.ones((LANES,), jnp.float32)

    def chunk(j, _):
        def sub(k, _2):
            idx = idxbuf[j, pl.ds(k * LANES, LANES)]
            plsc.addupdate_scatter(deg, [idx], ones)
            return 0
        lax.fori_loop(0, CHUNK // LANES, sub, 0)
        return 0
    lax.fori_loop(0, CPW, chunk, 0)

    pltpu.sync_copy(deg, out_hbm.at[w])


_hist = functools.partial(
    pl.kernel,
    out_type=jax.ShapeDtypeStruct((NW, N_ACC), jnp.float32),
    mesh=_mesh(),
    scratch_types=[
        pltpu.VMEM((CPW, CHUNK), jnp.int32),
        pltpu.VMEM((N_ACC,), jnp.float32),
    ],
    compiler_params=pltpu.CompilerParams(needs_layout_passes=False),
)(_hist_body)


def _scat_body(y_hbm, src_hbm, dst_hbm, out_hbm, sbuf, dbuf, rows_a, rows_b,
               acc, sem_a, sem_b):
    c = lax.axis_index("c")
    s = lax.axis_index("s")
    w = c * NS + s

    def zero(i, _):
        for jj in range(D // LANES):
            rows_a[i, pl.ds(jj * LANES, LANES)] = jnp.zeros((LANES,),
                                                            jnp.float32)
        return 0
    lax.fori_loop(0, CHUNK, zero, 0)
    for k in range(RPS // CHUNK):
        pltpu.sync_copy(rows_a, acc.at[pl.ds(s * RPS + k * CHUNK, CHUNK)])

    plsc.subcore_barrier()

    def gather(j, buf, sem):
        pltpu.async_copy(y_hbm.at[sbuf.at[j]], buf, sem)

    def drain(j, buf, sem):
        pltpu.make_async_copy(y_hbm.at[sbuf.at[j]], buf, sem).wait()

    def scat(j, buf):
        pltpu.sync_copy(buf, acc.at[dbuf.at[j]], add=True)

    def pipeline(row0, n_chunks):
        full = min(STAGE, n_chunks)
        sizes = [full] * (n_chunks // full)
        if n_chunks % full:
            sizes.append(n_chunks % full)
        offs = [sum(sizes[:i]) for i in range(len(sizes))]
        for off, stage in zip(offs, sizes):
            base = row0 + off
            pltpu.sync_copy(src_hbm.at[pl.ds(base, stage)],
                            sbuf.at[pl.ds(0, stage)])
            pltpu.sync_copy(dst_hbm.at[pl.ds(base, stage)],
                            dbuf.at[pl.ds(0, stage)])
            gather(0, rows_a, sem_a)

            def pair(i, _):
                ja = 2 * i
                jb = 2 * i + 1
                gather(jb, rows_b, sem_b)
                drain(ja, rows_a, sem_a)
                scat(ja, rows_a)
                gather(ja + 2, rows_a, sem_a)
                drain(jb, rows_b, sem_b)
                scat(jb, rows_b)
                return 0
            lax.fori_loop(0, (stage - 2) // 2, pair, 0)

            gather(stage - 1, rows_b, sem_b)
            drain(stage - 2, rows_a, sem_a)
            scat(stage - 2, rows_a)
            drain(stage - 1, rows_b, sem_b)
            scat(stage - 1, rows_b)

    @pl.when(c == 0)
    def _():
        pipeline(s * CPW0, CPW0)

    @pl.when(c == 1)
    def _():
        pipeline(NS * CPW0 + s * CPW1, CPW1)

    plsc.subcore_barrier()
    pltpu.sync_copy(acc.at[pl.ds(s * RPS, RPS)],
                    out_hbm.at[c, pl.ds(s * RPS, RPS)])


_scat = functools.partial(
    pl.kernel,
    out_type=jax.ShapeDtypeStruct((NC, N_ACC, D), jnp.float32),
    mesh=_mesh(),
    scratch_types=[
        pltpu.VMEM((STAGE, CHUNK), jnp.int32),
        pltpu.VMEM((STAGE, CHUNK), jnp.int32),
        pltpu.VMEM((CHUNK, D), jnp.float32),
        pltpu.VMEM((CHUNK, D), jnp.float32),
        pltpu.VMEM_SHARED((N_ACC, D), jnp.float32),
        pltpu.SemaphoreType.DMA,
        pltpu.SemaphoreType.DMA,
    ],
)(_scat_body)


def _mid_body(x_ref, w_ref, degp_ref, y_ref):
    xw = jnp.dot(x_ref[...], w_ref[...], preferred_element_type=jnp.float32)
    deg = jnp.sum(degp_ref[...], axis=1, keepdims=True) + 1.0
    y_ref[...] = xw * lax.rsqrt(deg)


_mid = pl.pallas_call(
    _mid_body,
    grid=(N // BLK,),
    in_specs=[
        pl.BlockSpec((BLK, D), lambda i: (i, 0)),
        pl.BlockSpec((D, D), lambda i: (0, 0)),
        pl.BlockSpec((BLK, NW), lambda i: (i, 0)),
    ],
    out_specs=pl.BlockSpec((BLK, D), lambda i: (i, 0)),
    out_shape=jax.ShapeDtypeStruct((N, D), jnp.float32),
)


def _epi_body(aggp_ref, y_ref, x_ref, degp_ref, b_ref, g_ref, bt_ref, o_ref):
    deg = jnp.sum(degp_ref[...], axis=1, keepdims=True) + 1.0
    dinv = lax.rsqrt(deg)
    t = dinv * (aggp_ref[0] + aggp_ref[1] + y_ref[...]) + b_ref[...] + x_ref[...]
    mu = jnp.mean(t, axis=-1, keepdims=True)
    var = jnp.mean((t - mu) ** 2, axis=-1, keepdims=True)
    t = (t - mu) * lax.rsqrt(var + 1e-5) * g_ref[...] + bt_ref[...]
    o_ref[...] = jnp.maximum(t, 0.0)


_epi = pl.pallas_call(
    _epi_body,
    grid=(N // BLK,),
    in_specs=[
        pl.BlockSpec((NC, BLK, D), lambda i: (0, i, 0)),
        pl.BlockSpec((BLK, D), lambda i: (i, 0)),
        pl.BlockSpec((BLK, D), lambda i: (i, 0)),
        pl.BlockSpec((BLK, NW), lambda i: (i, 0)),
        pl.BlockSpec((1, D), lambda i: (0, 0)),
        pl.BlockSpec((1, D), lambda i: (0, 0)),
        pl.BlockSpec((1, D), lambda i: (0, 0)),
    ],
    out_specs=pl.BlockSpec((BLK, D), lambda i: (i, 0)),
    out_shape=jax.ShapeDtypeStruct((N, D), jnp.float32),
)


def kernel(x, edge_index, W, b, gamma, beta):
    src = edge_index[0]
    dst = edge_index[1]
    pad = E_PAD - E
    src_p = jnp.concatenate(
        [src, jnp.zeros((pad,), jnp.int32)]).reshape(NW * CPW, CHUNK)
    dst_p = jnp.concatenate(
        [dst, jnp.full((pad,), N, jnp.int32)]).reshape(NW * CPW, CHUNK)
    degp = _hist(dst_p).T
    y = _mid(x, W, degp)
    aggp = _scat(y, src_p, dst_p)
    return _epi(aggp, y, x, degp,
                b.reshape(1, D), gamma.reshape(1, D), beta.reshape(1, D))


# speedup vs baseline: 15.3256x; 1.0147x over previous
import functools

import jax
import jax.numpy as jnp
from jax import lax
from jax.experimental import pallas as pl
from jax.experimental.pallas import tpu as pltpu
from jax.experimental.pallas import tpu_sc as plsc

N = 10000
D = 128
E = 320000
NC = 2
NS = 16
LANES = 16
NW = NC * NS
CHUNK = 128
CPW = 80
CPW0 = 152
CPW1 = 8
SSZ = 8
EPW = CPW * CHUNK
E_PAD = NW * EPW
N_ACC = 10240
RPS = N_ACC // NS
BLK = 1000


def _mesh():
    return plsc.VectorSubcoreMesh(
        core_axis_name="c", subcore_axis_name="s",
        num_cores=NC, num_subcores=NS)


def _hist_body(dst_hbm, out_hbm, idxbuf, deg):
    c = lax.axis_index("c")
    s = lax.axis_index("s")
    w = c * NS + s

    def zero(i, _):
        deg[pl.ds(i * LANES, LANES)] = jnp.zeros((LANES,), jnp.float32)
        return 0
    lax.fori_loop(0, N_ACC // LANES, zero, 0)

    pltpu.sync_copy(dst_hbm.at[pl.ds(w * CPW, CPW)], idxbuf)
    ones = jnp.ones((LANES,), jnp.float32)

    def chunk(j, _):
        def sub(k, _2):
            idx = idxbuf[j, pl.ds(k * LANES, LANES)]
            plsc.addupdate_scatter(deg, [idx], ones)
            return 0
        lax.fori_loop(0, CHUNK // LANES, sub, 0)
        return 0
    lax.fori_loop(0, CPW, chunk, 0)

    pltpu.sync_copy(deg, out_hbm.at[w])


_hist = functools.partial(
    pl.kernel,
    out_type=jax.ShapeDtypeStruct((NW, N_ACC), jnp.float32),
    mesh=_mesh(),
    scratch_types=[
        pltpu.VMEM((CPW, CHUNK), jnp.int32),
        pltpu.VMEM((N_ACC,), jnp.float32),
    ],
    compiler_params=pltpu.CompilerParams(needs_layout_passes=False),
)(_hist_body)


def _scat_body(y_hbm, src_hbm, dst_hbm, out_hbm, sbuf, dbuf, rows_a, rows_b,
               acc, sem_a, sem_b):
    c = lax.axis_index("c")
    s = lax.axis_index("s")
    w = c * NS + s

    def zero(i, _):
        for jj in range(D // LANES):
            rows_a[i, pl.ds(jj * LANES, LANES)] = jnp.zeros((LANES,),
                                                            jnp.float32)
        return 0
    lax.fori_loop(0, CHUNK, zero, 0)
    for k in range(RPS // CHUNK):
        pltpu.sync_copy(rows_a, acc.at[pl.ds(s * RPS + k * CHUNK, CHUNK)])

    plsc.subcore_barrier()

    def gather(j, buf, sem):
        pltpu.async_copy(y_hbm.at[sbuf.at[j]], buf, sem)

    def drain(j, buf, sem):
        pltpu.make_async_copy(y_hbm.at[sbuf.at[j]], buf, sem).wait()

    def scat(j, buf):
        pltpu.sync_copy(buf, acc.at[dbuf.at[j]], add=True)

    row0 = jnp.where(c == 0, s * CPW0, NS * CPW0 + s * CPW1)
    nstages = jnp.where(c == 0, CPW0 // SSZ, CPW1 // SSZ)

    def stage_fn(h, _):
        base = row0 + h * SSZ
        pltpu.sync_copy(src_hbm.at[pl.ds(base, SSZ)], sbuf)
        pltpu.sync_copy(dst_hbm.at[pl.ds(base, SSZ)], dbuf)
        gather(0, rows_a, sem_a)

        def pair(i, _2):
            ja = 2 * i
            jb = 2 * i + 1
            gather(jb, rows_b, sem_b)
            drain(ja, rows_a, sem_a)
            scat(ja, rows_a)
            gather(ja + 2, rows_a, sem_a)
            drain(jb, rows_b, sem_b)
            scat(jb, rows_b)
            return 0
        lax.fori_loop(0, (SSZ - 2) // 2, pair, 0)

        gather(SSZ - 1, rows_b, sem_b)
        drain(SSZ - 2, rows_a, sem_a)
        scat(SSZ - 2, rows_a)
        drain(SSZ - 1, rows_b, sem_b)
        scat(SSZ - 1, rows_b)
        return 0
    lax.fori_loop(0, nstages, stage_fn, 0)

    plsc.subcore_barrier()
    pltpu.sync_copy(acc.at[pl.ds(s * RPS, RPS)],
                    out_hbm.at[c, pl.ds(s * RPS, RPS)])


_scat = functools.partial(
    pl.kernel,
    out_type=jax.ShapeDtypeStruct((NC, N_ACC, D), jnp.float32),
    mesh=_mesh(),
    scratch_types=[
        pltpu.VMEM((SSZ, CHUNK), jnp.int32),
        pltpu.VMEM((SSZ, CHUNK), jnp.int32),
        pltpu.VMEM((CHUNK, D), jnp.float32),
        pltpu.VMEM((CHUNK, D), jnp.float32),
        pltpu.VMEM_SHARED((N_ACC, D), jnp.float32),
        pltpu.SemaphoreType.DMA,
        pltpu.SemaphoreType.DMA,
    ],
)(_scat_body)


def _mid_body(x_ref, w_ref, degp_ref, y_ref):
    xw = jnp.dot(x_ref[...], w_ref[...], preferred_element_type=jnp.float32)
    deg = jnp.sum(degp_ref[...], axis=1, keepdims=True) + 1.0
    y_ref[...] = xw * lax.rsqrt(deg)


_mid = pl.pallas_call(
    _mid_body,
    grid=(N // BLK,),
    in_specs=[
        pl.BlockSpec((BLK, D), lambda i: (i, 0)),
        pl.BlockSpec((D, D), lambda i: (0, 0)),
        pl.BlockSpec((BLK, NW), lambda i: (i, 0)),
    ],
    out_specs=pl.BlockSpec((BLK, D), lambda i: (i, 0)),
    out_shape=jax.ShapeDtypeStruct((N, D), jnp.float32),
)


def _epi_body(aggp_ref, y_ref, x_ref, degp_ref, b_ref, g_ref, bt_ref, o_ref):
    deg = jnp.sum(degp_ref[...], axis=1, keepdims=True) + 1.0
    dinv = lax.rsqrt(deg)
    t = dinv * (aggp_ref[0] + aggp_ref[1] + y_ref[...]) + b_ref[...] + x_ref[...]
    mu = jnp.mean(t, axis=-1, keepdims=True)
    var = jnp.mean((t - mu) ** 2, axis=-1, keepdims=True)
    t = (t - mu) * lax.rsqrt(var + 1e-5) * g_ref[...] + bt_ref[...]
    o_ref[...] = jnp.maximum(t, 0.0)


_epi = pl.pallas_call(
    _epi_body,
    grid=(N // BLK,),
    in_specs=[
        pl.BlockSpec((NC, BLK, D), lambda i: (0, i, 0)),
        pl.BlockSpec((BLK, D), lambda i: (i, 0)),
        pl.BlockSpec((BLK, D), lambda i: (i, 0)),
        pl.BlockSpec((BLK, NW), lambda i: (i, 0)),
        pl.BlockSpec((1, D), lambda i: (0, 0)),
        pl.BlockSpec((1, D), lambda i: (0, 0)),
        pl.BlockSpec((1, D), lambda i: (0, 0)),
    ],
    out_specs=pl.BlockSpec((BLK, D), lambda i: (i, 0)),
    out_shape=jax.ShapeDtypeStruct((N, D), jnp.float32),
)


def kernel(x, edge_index, W, b, gamma, beta):
    src = edge_index[0]
    dst = edge_index[1]
    pad = E_PAD - E
    src_p = jnp.concatenate(
        [src, jnp.zeros((pad,), jnp.int32)]).reshape(NW * CPW, CHUNK)
    dst_p = jnp.concatenate(
        [dst, jnp.full((pad,), N, jnp.int32)]).reshape(NW * CPW, CHUNK)
    degp = _hist(dst_p).T
    y = _mid(x, W, degp)
    aggp = _scat(y, src_p, dst_p)
    return _epi(aggp, y, x, degp,
                b.reshape(1, D), gamma.reshape(1, D), beta.reshape(1, D))
